# baseline (speedup 1.0000x reference)
import functools

import jax
import jax.numpy as jnp
from jax import lax
from jax.experimental import pallas as pl
from jax.experimental.pallas import tpu as pltpu
from jax.experimental.pallas import tpu_sc as plsc

N = 10000
E = 320000
D = 128
H = 128
C = 16
NLAYER = 4
ALPHA = 0.9
GAMMA = 0.1
LAMDA = 0.5

NC = 2
NS = 16
NW = NC * NS
EPT = E // NW
CHUNK = 80
NCHUNK = EPT // CHUNK
NG = 5
GC = NCHUNK // NG
NGD = 25
GCD = NCHUNK // NGD
FSL = D // 16
CP = N // NS - 1
TAIL = N - NS * CP
NBLK = N // CHUNK
ZPASS = -(-NBLK // NS)

_mesh = plsc.VectorSubcoreMesh(core_axis_name="c", subcore_axis_name="s")


def _degs_body(src_hbm, dst_hbm, w_hbm, degr_hbm, degc_hbm,
               idx, w_l, rows, degsh):
    c = lax.axis_index("c")
    s = lax.axis_index("s")
    wid = c * NS + s

    zero16 = jnp.zeros((16,), jnp.float32)

    def one_pass(edge_hbm, out_hbm):
        def _zrow(i, carry):
            for f in range(FSL):
                rows[i, pl.ds(f * 16, 16)] = zero16
            return carry
        lax.fori_loop(0, CHUNK, _zrow, 0)

        def _zacc(t, carry):
            b = t * NS + s

            @pl.when(b < NBLK)
            def _do():
                pltpu.sync_copy(rows, degsh.at[pl.ds(b * CHUNK, CHUNK)])
            return carry
        lax.fori_loop(0, ZPASS, _zacc, 0)

        plsc.subcore_barrier()

        def _gloop(g, carry0):
            pltpu.sync_copy(edge_hbm.at[wid, g], idx)
            pltpu.sync_copy(w_hbm.at[wid, g], w_l)

            def _chunk(j, carry):
                def _group(gg, carry2):
                    wvec = w_l[j, pl.ds(gg * 16, 16)]
                    for i in range(16):
                        k = gg * 16 + i
                        rows[k, pl.ds(0, 16)] = jnp.full(
                            (16,), wvec[i], dtype=jnp.float32)
                    return carry2
                lax.fori_loop(0, CHUNK // 16, _group, 0)

                pltpu.sync_copy(rows, degsh.at[idx.at[j]], add=True)
                return carry
            lax.fori_loop(0, GC, _chunk, 0)
            return carry0
        lax.fori_loop(0, NG, _gloop, 0)

        plsc.subcore_barrier()

        pltpu.sync_copy(degsh.at[pl.ds(s * CP, CP)],
                        out_hbm.at[c, pl.ds(s * CP, CP)])

        @pl.when(s == 0)
        def _out_tail():
            pltpu.sync_copy(degsh.at[pl.ds(NS * CP, TAIL)],
                            out_hbm.at[c, pl.ds(NS * CP, TAIL)])

        plsc.subcore_barrier()

    one_pass(src_hbm, degr_hbm)
    one_pass(dst_hbm, degc_hbm)


_degs = functools.partial(
    pl.kernel,
    out_type=(jax.ShapeDtypeStruct((NC, N, D), jnp.float32),
              jax.ShapeDtypeStruct((NC, N, D), jnp.float32)),
    mesh=_mesh,
    scratch_types=[
        pltpu.VMEM((GC, CHUNK), jnp.int32),
        pltpu.VMEM((GC, CHUNK), jnp.float32),
        pltpu.VMEM((CHUNK, D), jnp.float32),
        pltpu.VMEM_SHARED((N, D), jnp.float32),
    ],
)(_degs_body)


def _spmm_body(src_hbm, dst_hbm, w_hbm, table_hbm, out_hbm,
               idx_s, idx_d, w_l, rows, acc, sem):
    c = lax.axis_index("c")
    s = lax.axis_index("s")
    wid = c * NS + s

    zero16 = jnp.zeros((16,), jnp.float32)

    def _zrow(i, carry):
        for f in range(FSL):
            rows[0, i, pl.ds(f * 16, 16)] = zero16
        return carry
    lax.fori_loop(0, CHUNK, _zrow, 0)

    def _zacc(t, carry):
        b = t * NS + s

        @pl.when(b < NBLK)
        def _do():
            pltpu.sync_copy(rows.at[0], acc.at[pl.ds(b * CHUNK, CHUNK)])
        return carry
    lax.fori_loop(0, ZPASS, _zacc, 0)

    plsc.subcore_barrier()

    def _gloop(g, carry0):
        pltpu.sync_copy(src_hbm.at[wid, g], idx_s)
        pltpu.sync_copy(dst_hbm.at[wid, g], idx_d)
        pltpu.sync_copy(w_hbm.at[wid, g], w_l)

        pltpu.make_async_copy(table_hbm.at[idx_s.at[0]], rows.at[0],
                              sem.at[0]).start()

        def _chunk(j, carry):
            p = j % 2
            nx = j + 1

            @pl.when(nx < GC)
            def _prefetch():
                pltpu.make_async_copy(table_hbm.at[idx_s.at[nx]],
                                      rows.at[nx % 2], sem.at[nx % 2]).start()

            pltpu.make_async_copy(table_hbm.at[idx_s.at[j]], rows.at[p],
                                  sem.at[p]).wait()

            def _group(gg, carry2):
                wvec = w_l[j, pl.ds(gg * 16, 16)]
                for i in range(16):
                    k = gg * 16 + i
                    wv = jnp.full((16,), wvec[i], dtype=jnp.float32)
                    for f in range(FSL):
                        sl = pl.ds(f * 16, 16)
                        rows[p, k, sl] = rows[p, k, sl] * wv
                return carry2
            lax.fori_loop(0, CHUNK // 16, _group, 0)

            pltpu.sync_copy(rows.at[p], acc.at[idx_d.at[j]], add=True)
            return carry
        lax.fori_loop(0, GC, _chunk, 0)
        return carry0
    lax.fori_loop(0, NG, _gloop, 0)

    plsc.subcore_barrier()

    pltpu.sync_copy(acc.at[pl.ds(s * CP, CP)], out_hbm.at[c, pl.ds(s * CP, CP)])

    @pl.when(s == 0)
    def _out_tail():
        pltpu.sync_copy(acc.at[pl.ds(NS * CP, TAIL)],
                        out_hbm.at[c, pl.ds(NS * CP, TAIL)])


_spmm = functools.partial(
    pl.kernel,
    out_type=jax.ShapeDtypeStruct((NC, N, D), jnp.float32),
    mesh=_mesh,
    scratch_types=[
        pltpu.VMEM((GC, CHUNK), jnp.int32),
        pltpu.VMEM((GC, CHUNK), jnp.int32),
        pltpu.VMEM((GC, CHUNK), jnp.float32),
        pltpu.VMEM((2, CHUNK, D), jnp.float32),
        pltpu.VMEM_SHARED((N, D), jnp.float32),
        pltpu.SemaphoreType.DMA((2,)),
    ],
)(_spmm_body)


BR = 1000
GRID = N // BR

_PREC = lax.Precision.HIGHEST


def _tc_front_kern(x_ref, ax0_ref, ax1_ref, dg_ref, dinv_ref,
                   W0_ref, b0_ref, Whp_ref, bhp_ref,
                   h_ref, lx_ref, hs_ref):
    x = x_ref[...]
    h = jnp.maximum(
        jnp.dot(x, W0_ref[...], precision=_PREC,
                preferred_element_type=jnp.float32) + b0_ref[...], 0.0)
    h_ref[...] = h
    hs_ref[...] = dinv_ref[...] * h
    t = dg_ref[...] * x - ax0_ref[...] - ax1_ref[...]
    lx_ref[...] = jnp.maximum(
        jnp.dot(t, Whp_ref[...], precision=_PREC,
                preferred_element_type=jnp.float32) + bhp_ref[...], 0.0)


def _tc_front(x, ax0, ax1, dg, dinv2d, W0, b0, Whp, bhp):
    row_spec = pl.BlockSpec((BR, D), lambda i: (i, 0))
    col_spec = pl.BlockSpec((BR, 1), lambda i: (i, 0))
    return pl.pallas_call(
        _tc_front_kern,
        grid=(GRID,),
        in_specs=[row_spec, row_spec, row_spec, col_spec, col_spec,
                  pl.BlockSpec((D, H), lambda i: (0, 0)),
                  pl.BlockSpec((1, H), lambda i: (0, 0)),
                  pl.BlockSpec((D, H), lambda i: (0, 0)),
                  pl.BlockSpec((1, H), lambda i: (0, 0))],
        out_specs=[pl.BlockSpec((BR, H), lambda i: (i, 0)),
                   pl.BlockSpec((BR, H), lambda i: (i, 0)),
                   pl.BlockSpec((BR, H), lambda i: (i, 0))],
        out_shape=[jax.ShapeDtypeStruct((N, H), jnp.float32),
                   jax.ShapeDtypeStruct((N, H), jnp.float32),
                   jax.ShapeDtypeStruct((N, H), jnp.float32)],
    )(x, ax0, ax1, dg, dinv2d, W0, b0, Whp, bhp)


def _tc_layer_kern(beta, agg0_ref, agg1_ref, h_ref, h0_ref, lx_ref,
                   dinv_ref, Wg_ref, bg_ref, out_ref, outs_ref):
    dv = dinv_ref[...]
    agg = dv * (agg0_ref[...] + agg1_ref[...]) + dv * dv * h_ref[...]
    support = (ALPHA * agg + (1.0 - ALPHA) * h0_ref[...]
               + GAMMA * lx_ref[...])
    hn = jnp.maximum(
        (1.0 - beta) * support
        + beta * (jnp.dot(support, Wg_ref[...], precision=_PREC,
                          preferred_element_type=jnp.float32) + bg_ref[...]),
        0.0)
    out_ref[...] = hn
    outs_ref[...] = dv * hn


def _tc_layer(beta, agg0, agg1, h, h0, lx, dinv2d, Wgi, bgi):
    row_spec = pl.BlockSpec((BR, H), lambda i: (i, 0))
    return pl.pallas_call(
        functools.partial(_tc_layer_kern, beta),
        grid=(GRID,),
        in_specs=[row_spec, row_spec, row_spec, row_spec, row_spec,
                  pl.BlockSpec((BR, 1), lambda i: (i, 0)),
                  pl.BlockSpec((H, H), lambda i: (0, 0)),
                  pl.BlockSpec((1, H), lambda i: (0, 0))],
        out_specs=[row_spec, row_spec],
        out_shape=[jax.ShapeDtypeStruct((N, H), jnp.float32),
                   jax.ShapeDtypeStruct((N, H), jnp.float32)],
    )(agg0, agg1, h, h0, lx, dinv2d, Wgi, bgi)


def _tc_out_kern(h_ref, Wout_ref, bout_ref, o_ref):
    z = jnp.dot(h_ref[...], Wout_ref[...], precision=_PREC,
                preferred_element_type=jnp.float32) + bout_ref[...]
    m = jnp.max(z, axis=1, keepdims=True)
    zs = z - m
    o_ref[...] = zs - jnp.log(jnp.sum(jnp.exp(zs), axis=1, keepdims=True))


def _tc_out(h, Wout, bout):
    return pl.pallas_call(
        _tc_out_kern,
        grid=(GRID,),
        in_specs=[pl.BlockSpec((BR, H), lambda i: (i, 0)),
                  pl.BlockSpec((H, C), lambda i: (0, 0)),
                  pl.BlockSpec((1, C), lambda i: (0, 0))],
        out_specs=pl.BlockSpec((BR, C), lambda i: (i, 0)),
        out_shape=jax.ShapeDtypeStruct((N, C), jnp.float32),
    )(h, Wout, bout)


def kernel(x, edge_index, edge_weight, W0, b0, Whp, bhp, Wg, bg, Wout, bout):
    row = edge_index[0]
    col = edge_index[1]
    src3 = row.reshape(NW, NG, GC, CHUNK)
    dst3 = col.reshape(NW, NG, GC, CHUNK)
    w3 = edge_weight.reshape(NW, NG, GC, CHUNK)
    axp = _spmm(src3, dst3, w3, x)
    degr, degc = _degs(src3, dst3, w3)
    deg_row = (degr[0, :, 0] + degr[1, :, 0]).reshape(N, 1)
    dinv = lax.rsqrt(degc[0, :, 0] + degc[1, :, 0] + 1.0)
    dinv2d = dinv.reshape(N, 1)
    h, lx, hs = _tc_front(x, axp[0], axp[1], deg_row, dinv2d,
                          W0, b0.reshape(1, H), Whp, bhp.reshape(1, H))
    h0 = h
    for i in range(NLAYER):
        beta = LAMDA / (i + 2)
        aggp = _spmm(src3, dst3, w3, hs)
        h, hs = _tc_layer(beta, aggp[0], aggp[1], h, h0, lx, dinv2d,
                          Wg[i], bg[i].reshape(1, H))
    return _tc_out(h, Wout, bout.reshape(1, C))


# speedup vs baseline: 1.5687x; 1.5687x over previous
import functools

import jax
import jax.numpy as jnp
from jax import lax
from jax.experimental import pallas as pl
from jax.experimental.pallas import tpu as pltpu
from jax.experimental.pallas import tpu_sc as plsc

N = 10000
E = 320000
D = 128
H = 128
C = 16
NLAYER = 4
ALPHA = 0.9
GAMMA = 0.1
LAMDA = 0.5

NC = 2
NS = 16
NW = NC * NS
EPT = E // NW
CHUNK = 80
NCHUNK = EPT // CHUNK
NG = 5
GC = NCHUNK // NG
NGD = 25
GCD = NCHUNK // NGD
FSL = D // 16
CP = N // NS - 1
TAIL = N - NS * CP
NBLK = N // CHUNK
ZPASS = -(-NBLK // NS)

_mesh = plsc.VectorSubcoreMesh(core_axis_name="c", subcore_axis_name="s")


def _degs_body(src_hbm, dst_hbm, w_hbm, degr_hbm, degc_hbm,
               idx, w_l, rows, degsh):
    c = lax.axis_index("c")
    s = lax.axis_index("s")
    wid = c * NS + s

    zero16 = jnp.zeros((16,), jnp.float32)

    def one_pass(edge_hbm, out_hbm):
        def _zrow(i, carry):
            for f in range(FSL):
                rows[i, pl.ds(f * 16, 16)] = zero16
            return carry
        lax.fori_loop(0, CHUNK, _zrow, 0)

        def _zacc(t, carry):
            b = t * NS + s

            @pl.when(b < NBLK)
            def _do():
                pltpu.sync_copy(rows, degsh.at[pl.ds(b * CHUNK, CHUNK)])
            return carry
        lax.fori_loop(0, ZPASS, _zacc, 0)

        plsc.subcore_barrier()

        def _gloop(g, carry0):
            pltpu.sync_copy(edge_hbm.at[wid, g], idx)
            pltpu.sync_copy(w_hbm.at[wid, g], w_l)

            def _chunk(j, carry):
                def _group(gg, carry2):
                    wvec = w_l[j, pl.ds(gg * 16, 16)]
                    for i in range(16):
                        k = gg * 16 + i
                        rows[k, pl.ds(0, 16)] = jnp.full(
                            (16,), wvec[i], dtype=jnp.float32)
                    return carry2
                lax.fori_loop(0, CHUNK // 16, _group, 0)

                pltpu.sync_copy(rows, degsh.at[idx.at[j]], add=True)
                return carry
            lax.fori_loop(0, GC, _chunk, 0)
            return carry0
        lax.fori_loop(0, NG, _gloop, 0)

        plsc.subcore_barrier()

        pltpu.sync_copy(degsh.at[pl.ds(s * CP, CP)],
                        out_hbm.at[c, pl.ds(s * CP, CP)])

        @pl.when(s == 0)
        def _out_tail():
            pltpu.sync_copy(degsh.at[pl.ds(NS * CP, TAIL)],
                            out_hbm.at[c, pl.ds(NS * CP, TAIL)])

        plsc.subcore_barrier()

    one_pass(src_hbm, degr_hbm)
    one_pass(dst_hbm, degc_hbm)


_degs = functools.partial(
    pl.kernel,
    out_type=(jax.ShapeDtypeStruct((NC, N, D), jnp.float32),
              jax.ShapeDtypeStruct((NC, N, D), jnp.float32)),
    mesh=_mesh,
    scratch_types=[
        pltpu.VMEM((GC, CHUNK), jnp.int32),
        pltpu.VMEM((GC, CHUNK), jnp.float32),
        pltpu.VMEM((CHUNK, D), jnp.float32),
        pltpu.VMEM_SHARED((N, D), jnp.float32),
    ],
)(_degs_body)


def _spmm_body(src_hbm, dst_hbm, w_hbm, table_hbm, out_hbm,
               idx_s, idx_d, w_l, rows, acc, sem):
    c = lax.axis_index("c")
    s = lax.axis_index("s")
    wid = c * NS + s

    zero16 = jnp.zeros((16,), jnp.float32)

    def _zrow(i, carry):
        for f in range(FSL):
            rows[0, i, pl.ds(f * 16, 16)] = zero16
        return carry
    lax.fori_loop(0, CHUNK, _zrow, 0)

    def _zacc(t, carry):
        b = t * NS + s

        @pl.when(b < NBLK)
        def _do():
            pltpu.sync_copy(rows.at[0], acc.at[pl.ds(b * CHUNK, CHUNK)])
        return carry
    lax.fori_loop(0, ZPASS, _zacc, 0)

    plsc.subcore_barrier()

    def _gloop(g, carry0):
        pltpu.sync_copy(src_hbm.at[wid, g], idx_s)
        pltpu.sync_copy(dst_hbm.at[wid, g], idx_d)
        pltpu.sync_copy(w_hbm.at[wid, g], w_l)

        def _chunk(j, carry):
            pltpu.async_copy(table_hbm.at[idx_s.at[j]], rows.at[0],
                             sem.at[0]).wait()

            def _group(gg, carry2):
                wvec = w_l[j, pl.ds(gg * 16, 16)]
                for i in range(16):
                    k = gg * 16 + i
                    wv = jnp.full((16,), wvec[i], dtype=jnp.float32)
                    for f in range(FSL):
                        sl = pl.ds(f * 16, 16)
                        rows[0, k, sl] = rows[0, k, sl] * wv
                return carry2
            lax.fori_loop(0, CHUNK // 16, _group, 0)

            pltpu.sync_copy(rows.at[0], acc.at[idx_d.at[j]], add=True)
            return carry
        lax.fori_loop(0, GC, _chunk, 0)
        return carry0
    lax.fori_loop(0, NG, _gloop, 0)

    plsc.subcore_barrier()

    pltpu.sync_copy(acc.at[pl.ds(s * CP, CP)], out_hbm.at[c, pl.ds(s * CP, CP)])

    @pl.when(s == 0)
    def _out_tail():
        pltpu.sync_copy(acc.at[pl.ds(NS * CP, TAIL)],
                        out_hbm.at[c, pl.ds(NS * CP, TAIL)])


_spmm = functools.partial(
    pl.kernel,
    out_type=jax.ShapeDtypeStruct((NC, N, D), jnp.float32),
    mesh=_mesh,
    scratch_types=[
        pltpu.VMEM((GC, CHUNK), jnp.int32),
        pltpu.VMEM((GC, CHUNK), jnp.int32),
        pltpu.VMEM((GC, CHUNK), jnp.float32),
        pltpu.VMEM((2, CHUNK, D), jnp.float32),
        pltpu.VMEM_SHARED((N, D), jnp.float32),
        pltpu.SemaphoreType.DMA((2,)),
    ],
)(_spmm_body)


BR = 1000
GRID = N // BR

_PREC = lax.Precision.HIGHEST


def _tc_front_kern(x_ref, ax0_ref, ax1_ref, dg_ref, dinv_ref,
                   W0_ref, b0_ref, Whp_ref, bhp_ref,
                   h_ref, lx_ref, hs_ref):
    x = x_ref[...]
    h = jnp.maximum(
        jnp.dot(x, W0_ref[...], precision=_PREC,
                preferred_element_type=jnp.float32) + b0_ref[...], 0.0)
    h_ref[...] = h
    hs_ref[...] = dinv_ref[...] * h
    t = dg_ref[...] * x - ax0_ref[...] - ax1_ref[...]
    lx_ref[...] = jnp.maximum(
        jnp.dot(t, Whp_ref[...], precision=_PREC,
                preferred_element_type=jnp.float32) + bhp_ref[...], 0.0)


def _tc_front(x, ax0, ax1, dg, dinv2d, W0, b0, Whp, bhp):
    row_spec = pl.BlockSpec((BR, D), lambda i: (i, 0))
    col_spec = pl.BlockSpec((BR, 1), lambda i: (i, 0))
    return pl.pallas_call(
        _tc_front_kern,
        grid=(GRID,),
        in_specs=[row_spec, row_spec, row_spec, col_spec, col_spec,
                  pl.BlockSpec((D, H), lambda i: (0, 0)),
                  pl.BlockSpec((1, H), lambda i: (0, 0)),
                  pl.BlockSpec((D, H), lambda i: (0, 0)),
                  pl.BlockSpec((1, H), lambda i: (0, 0))],
        out_specs=[pl.BlockSpec((BR, H), lambda i: (i, 0)),
                   pl.BlockSpec((BR, H), lambda i: (i, 0)),
                   pl.BlockSpec((BR, H), lambda i: (i, 0))],
        out_shape=[jax.ShapeDtypeStruct((N, H), jnp.float32),
                   jax.ShapeDtypeStruct((N, H), jnp.float32),
                   jax.ShapeDtypeStruct((N, H), jnp.float32)],
    )(x, ax0, ax1, dg, dinv2d, W0, b0, Whp, bhp)


def _tc_layer_kern(beta, agg0_ref, agg1_ref, h_ref, h0_ref, lx_ref,
                   dinv_ref, Wg_ref, bg_ref, out_ref, outs_ref):
    dv = dinv_ref[...]
    agg = dv * (agg0_ref[...] + agg1_ref[...]) + dv * dv * h_ref[...]
    support = (ALPHA * agg + (1.0 - ALPHA) * h0_ref[...]
               + GAMMA * lx_ref[...])
    hn = jnp.maximum(
        (1.0 - beta) * support
        + beta * (jnp.dot(support, Wg_ref[...], precision=_PREC,
                          preferred_element_type=jnp.float32) + bg_ref[...]),
        0.0)
    out_ref[...] = hn
    outs_ref[...] = dv * hn


def _tc_layer(beta, agg0, agg1, h, h0, lx, dinv2d, Wgi, bgi):
    row_spec = pl.BlockSpec((BR, H), lambda i: (i, 0))
    return pl.pallas_call(
        functools.partial(_tc_layer_kern, beta),
        grid=(GRID,),
        in_specs=[row_spec, row_spec, row_spec, row_spec, row_spec,
                  pl.BlockSpec((BR, 1), lambda i: (i, 0)),
                  pl.BlockSpec((H, H), lambda i: (0, 0)),
                  pl.BlockSpec((1, H), lambda i: (0, 0))],
        out_specs=[row_spec, row_spec],
        out_shape=[jax.ShapeDtypeStruct((N, H), jnp.float32),
                   jax.ShapeDtypeStruct((N, H), jnp.float32)],
    )(agg0, agg1, h, h0, lx, dinv2d, Wgi, bgi)


def _tc_out_kern(h_ref, Wout_ref, bout_ref, o_ref):
    z = jnp.dot(h_ref[...], Wout_ref[...], precision=_PREC,
                preferred_element_type=jnp.float32) + bout_ref[...]
    m = jnp.max(z, axis=1, keepdims=True)
    zs = z - m
    o_ref[...] = zs - jnp.log(jnp.sum(jnp.exp(zs), axis=1, keepdims=True))


def _tc_out(h, Wout, bout):
    return pl.pallas_call(
        _tc_out_kern,
        grid=(GRID,),
        in_specs=[pl.BlockSpec((BR, H), lambda i: (i, 0)),
                  pl.BlockSpec((H, C), lambda i: (0, 0)),
                  pl.BlockSpec((1, C), lambda i: (0, 0))],
        out_specs=pl.BlockSpec((BR, C), lambda i: (i, 0)),
        out_shape=jax.ShapeDtypeStruct((N, C), jnp.float32),
    )(h, Wout, bout)


def kernel(x, edge_index, edge_weight, W0, b0, Whp, bhp, Wg, bg, Wout, bout):
    row = edge_index[0]
    col = edge_index[1]
    src3 = row.reshape(NW, NG, GC, CHUNK)
    dst3 = col.reshape(NW, NG, GC, CHUNK)
    w3 = edge_weight.reshape(NW, NG, GC, CHUNK)
    axp = _spmm(src3, dst3, w3, x)
    degr, degc = _degs(src3, dst3, w3)
    deg_row = (degr[0, :, 0] + degr[1, :, 0]).reshape(N, 1)
    dinv = lax.rsqrt(degc[0, :, 0] + degc[1, :, 0] + 1.0)
    dinv2d = dinv.reshape(N, 1)
    h, lx, hs = _tc_front(x, axp[0], axp[1], deg_row, dinv2d,
                          W0, b0.reshape(1, H), Whp, bhp.reshape(1, H))
    h0 = h
    for i in range(NLAYER):
        beta = LAMDA / (i + 2)
        aggp = _spmm(src3, dst3, w3, hs)
        h, hs = _tc_layer(beta, aggp[0], aggp[1], h, h0, lx, dinv2d,
                          Wg[i], bg[i].reshape(1, H))
    return _tc_out(h, Wout, bout.reshape(1, C))


# speedup vs baseline: 1.5783x; 1.0061x over previous
import functools

import jax
import jax.numpy as jnp
from jax import lax
from jax.experimental import pallas as pl
from jax.experimental.pallas import tpu as pltpu
from jax.experimental.pallas import tpu_sc as plsc

N = 10000
E = 320000
D = 128
H = 128
C = 16
NLAYER = 4
ALPHA = 0.9
GAMMA = 0.1
LAMDA = 0.5

NC = 2
NS = 16
NW = NC * NS
EPT = E // NW
CHUNK = 80
NCHUNK = EPT // CHUNK
NG = 5
GC = NCHUNK // NG
NGD = 25
GCD = NCHUNK // NGD
FSL = D // 16
CP = N // NS - 1
TAIL = N - NS * CP
NBLK = N // CHUNK
ZPASS = -(-NBLK // NS)

_mesh = plsc.VectorSubcoreMesh(core_axis_name="c", subcore_axis_name="s")


def _degs_body(src_hbm, dst_hbm, w_hbm, degr_hbm, degc_hbm,
               idx, w_l, rows, degsh):
    c = lax.axis_index("c")
    s = lax.axis_index("s")
    wid = c * NS + s

    zero16 = jnp.zeros((16,), jnp.float32)

    def one_pass(edge_hbm, out_hbm):
        def _zrow(i, carry):
            for f in range(FSL):
                rows[i, pl.ds(f * 16, 16)] = zero16
            return carry
        lax.fori_loop(0, CHUNK, _zrow, 0)

        def _zacc(t, carry):
            b = t * NS + s

            @pl.when(b < NBLK)
            def _do():
                pltpu.sync_copy(rows, degsh.at[pl.ds(b * CHUNK, CHUNK)])
            return carry
        lax.fori_loop(0, ZPASS, _zacc, 0)

        plsc.subcore_barrier()

        def _gloop(g, carry0):
            pltpu.sync_copy(edge_hbm.at[wid, g], idx)
            pltpu.sync_copy(w_hbm.at[wid, g], w_l)

            def _chunk(j, carry):
                def _group(gg, carry2):
                    wvec = w_l[j, pl.ds(gg * 16, 16)]
                    for i in range(16):
                        k = gg * 16 + i
                        rows[k, pl.ds(0, 16)] = jnp.full(
                            (16,), wvec[i], dtype=jnp.float32)
                    return carry2
                lax.fori_loop(0, CHUNK // 16, _group, 0)

                pltpu.sync_copy(rows, degsh.at[idx.at[j]], add=True)
                return carry
            lax.fori_loop(0, GC, _chunk, 0)
            return carry0
        lax.fori_loop(0, NG, _gloop, 0)

        plsc.subcore_barrier()

        pltpu.sync_copy(degsh.at[pl.ds(s * CP, CP)],
                        out_hbm.at[c, pl.ds(s * CP, CP)])

        @pl.when(s == 0)
        def _out_tail():
            pltpu.sync_copy(degsh.at[pl.ds(NS * CP, TAIL)],
                            out_hbm.at[c, pl.ds(NS * CP, TAIL)])

        plsc.subcore_barrier()

    one_pass(src_hbm, degr_hbm)
    one_pass(dst_hbm, degc_hbm)


_degs = functools.partial(
    pl.kernel,
    out_type=(jax.ShapeDtypeStruct((NC, N, D), jnp.float32),
              jax.ShapeDtypeStruct((NC, N, D), jnp.float32)),
    mesh=_mesh,
    scratch_types=[
        pltpu.VMEM((GC, CHUNK), jnp.int32),
        pltpu.VMEM((GC, CHUNK), jnp.float32),
        pltpu.VMEM((CHUNK, D), jnp.float32),
        pltpu.VMEM_SHARED((N, D), jnp.float32),
    ],
)(_degs_body)


def _spmm_body(src_hbm, dst_hbm, w_hbm, table_hbm, out_hbm,
               idx_s, idx_d, w_l, rows, acc, sem):
    c = lax.axis_index("c")
    s = lax.axis_index("s")
    wid = c * NS + s

    zero16 = jnp.zeros((16,), jnp.float32)

    def _zrow(i, carry):
        for f in range(FSL):
            rows[0, i, pl.ds(f * 16, 16)] = zero16
        return carry
    lax.fori_loop(0, CHUNK, _zrow, 0)

    def _zacc(t, carry):
        b = t * NS + s

        @pl.when(b < NBLK)
        def _do():
            pltpu.sync_copy(rows.at[0], acc.at[pl.ds(b * CHUNK, CHUNK)])
        return carry
    lax.fori_loop(0, ZPASS, _zacc, 0)

    plsc.subcore_barrier()

    def _gloop(g, carry0):
        pltpu.sync_copy(src_hbm.at[wid, g], idx_s)
        pltpu.sync_copy(dst_hbm.at[wid, g], idx_d)
        pltpu.sync_copy(w_hbm.at[wid, g], w_l)

        def _chunk(j, carry):
            pltpu.async_copy(table_hbm.at[idx_s.at[j]], rows.at[0],
                             sem.at[0]).wait()

            @plsc.parallel_loop(0, CHUNK // 16)
            def _group(gg):
                wvec = w_l[j, pl.ds(gg * 16, 16)]
                for i in range(16):
                    k = gg * 16 + i
                    wv = jnp.full((16,), wvec[i], dtype=jnp.float32)
                    for f in range(FSL):
                        sl = pl.ds(f * 16, 16)
                        rows[0, k, sl] = rows[0, k, sl] * wv

            pltpu.sync_copy(rows.at[0], acc.at[idx_d.at[j]], add=True)
            return carry
        lax.fori_loop(0, GC, _chunk, 0)
        return carry0
    lax.fori_loop(0, NG, _gloop, 0)

    plsc.subcore_barrier()

    pltpu.sync_copy(acc.at[pl.ds(s * CP, CP)], out_hbm.at[c, pl.ds(s * CP, CP)])

    @pl.when(s == 0)
    def _out_tail():
        pltpu.sync_copy(acc.at[pl.ds(NS * CP, TAIL)],
                        out_hbm.at[c, pl.ds(NS * CP, TAIL)])


_spmm = functools.partial(
    pl.kernel,
    out_type=jax.ShapeDtypeStruct((NC, N, D), jnp.float32),
    mesh=_mesh,
    scratch_types=[
        pltpu.VMEM((GC, CHUNK), jnp.int32),
        pltpu.VMEM((GC, CHUNK), jnp.int32),
        pltpu.VMEM((GC, CHUNK), jnp.float32),
        pltpu.VMEM((2, CHUNK, D), jnp.float32),
        pltpu.VMEM_SHARED((N, D), jnp.float32),
        pltpu.SemaphoreType.DMA((2,)),
    ],
)(_spmm_body)


BR = 1000
GRID = N // BR

_PREC = lax.Precision.HIGHEST


def _tc_front_kern(x_ref, ax0_ref, ax1_ref, dg_ref, dinv_ref,
                   W0_ref, b0_ref, Whp_ref, bhp_ref,
                   h_ref, lx_ref, hs_ref):
    x = x_ref[...]
    h = jnp.maximum(
        jnp.dot(x, W0_ref[...], precision=_PREC,
                preferred_element_type=jnp.float32) + b0_ref[...], 0.0)
    h_ref[...] = h
    hs_ref[...] = dinv_ref[...] * h
    t = dg_ref[...] * x - ax0_ref[...] - ax1_ref[...]
    lx_ref[...] = jnp.maximum(
        jnp.dot(t, Whp_ref[...], precision=_PREC,
                preferred_element_type=jnp.float32) + bhp_ref[...], 0.0)


def _tc_front(x, ax0, ax1, dg, dinv2d, W0, b0, Whp, bhp):
    row_spec = pl.BlockSpec((BR, D), lambda i: (i, 0))
    col_spec = pl.BlockSpec((BR, 1), lambda i: (i, 0))
    return pl.pallas_call(
        _tc_front_kern,
        grid=(GRID,),
        in_specs=[row_spec, row_spec, row_spec, col_spec, col_spec,
                  pl.BlockSpec((D, H), lambda i: (0, 0)),
                  pl.BlockSpec((1, H), lambda i: (0, 0)),
                  pl.BlockSpec((D, H), lambda i: (0, 0)),
                  pl.BlockSpec((1, H), lambda i: (0, 0))],
        out_specs=[pl.BlockSpec((BR, H), lambda i: (i, 0)),
                   pl.BlockSpec((BR, H), lambda i: (i, 0)),
                   pl.BlockSpec((BR, H), lambda i: (i, 0))],
        out_shape=[jax.ShapeDtypeStruct((N, H), jnp.float32),
                   jax.ShapeDtypeStruct((N, H), jnp.float32),
                   jax.ShapeDtypeStruct((N, H), jnp.float32)],
    )(x, ax0, ax1, dg, dinv2d, W0, b0, Whp, bhp)


def _tc_layer_kern(beta, agg0_ref, agg1_ref, h_ref, h0_ref, lx_ref,
                   dinv_ref, Wg_ref, bg_ref, out_ref, outs_ref):
    dv = dinv_ref[...]
    agg = dv * (agg0_ref[...] + agg1_ref[...]) + dv * dv * h_ref[...]
    support = (ALPHA * agg + (1.0 - ALPHA) * h0_ref[...]
               + GAMMA * lx_ref[...])
    hn = jnp.maximum(
        (1.0 - beta) * support
        + beta * (jnp.dot(support, Wg_ref[...], precision=_PREC,
                          preferred_element_type=jnp.float32) + bg_ref[...]),
        0.0)
    out_ref[...] = hn
    outs_ref[...] = dv * hn


def _tc_layer(beta, agg0, agg1, h, h0, lx, dinv2d, Wgi, bgi):
    row_spec = pl.BlockSpec((BR, H), lambda i: (i, 0))
    return pl.pallas_call(
        functools.partial(_tc_layer_kern, beta),
        grid=(GRID,),
        in_specs=[row_spec, row_spec, row_spec, row_spec, row_spec,
                  pl.BlockSpec((BR, 1), lambda i: (i, 0)),
                  pl.BlockSpec((H, H), lambda i: (0, 0)),
                  pl.BlockSpec((1, H), lambda i: (0, 0))],
        out_specs=[row_spec, row_spec],
        out_shape=[jax.ShapeDtypeStruct((N, H), jnp.float32),
                   jax.ShapeDtypeStruct((N, H), jnp.float32)],
    )(agg0, agg1, h, h0, lx, dinv2d, Wgi, bgi)


def _tc_out_kern(h_ref, Wout_ref, bout_ref, o_ref):
    z = jnp.dot(h_ref[...], Wout_ref[...], precision=_PREC,
                preferred_element_type=jnp.float32) + bout_ref[...]
    m = jnp.max(z, axis=1, keepdims=True)
    zs = z - m
    o_ref[...] = zs - jnp.log(jnp.sum(jnp.exp(zs), axis=1, keepdims=True))


def _tc_out(h, Wout, bout):
    return pl.pallas_call(
        _tc_out_kern,
        grid=(GRID,),
        in_specs=[pl.BlockSpec((BR, H), lambda i: (i, 0)),
                  pl.BlockSpec((H, C), lambda i: (0, 0)),
                  pl.BlockSpec((1, C), lambda i: (0, 0))],
        out_specs=pl.BlockSpec((BR, C), lambda i: (i, 0)),
        out_shape=jax.ShapeDtypeStruct((N, C), jnp.float32),
    )(h, Wout, bout)


def kernel(x, edge_index, edge_weight, W0, b0, Whp, bhp, Wg, bg, Wout, bout):
    row = edge_index[0]
    col = edge_index[1]
    src3 = row.reshape(NW, NG, GC, CHUNK)
    dst3 = col.reshape(NW, NG, GC, CHUNK)
    w3 = edge_weight.reshape(NW, NG, GC, CHUNK)
    axp = _spmm(src3, dst3, w3, x)
    degr, degc = _degs(src3, dst3, w3)
    deg_row = (degr[0, :, 0] + degr[1, :, 0]).reshape(N, 1)
    dinv = lax.rsqrt(degc[0, :, 0] + degc[1, :, 0] + 1.0)
    dinv2d = dinv.reshape(N, 1)
    h, lx, hs = _tc_front(x, axp[0], axp[1], deg_row, dinv2d,
                          W0, b0.reshape(1, H), Whp, bhp.reshape(1, H))
    h0 = h
    for i in range(NLAYER):
        beta = LAMDA / (i + 2)
        aggp = _spmm(src3, dst3, w3, hs)
        h, hs = _tc_layer(beta, aggp[0], aggp[1], h, h0, lx, dinv2d,
                          Wg[i], bg[i].reshape(1, H))
    return _tc_out(h, Wout, bout.reshape(1, C))


# speedup vs baseline: 2.3205x; 1.4703x over previous
import functools

import jax
import jax.numpy as jnp
from jax import lax
from jax.experimental import pallas as pl
from jax.experimental.pallas import tpu as pltpu
from jax.experimental.pallas import tpu_sc as plsc

N = 10000
E = 320000
D = 128
H = 128
C = 16
NLAYER = 4
ALPHA = 0.9
GAMMA = 0.1
LAMDA = 0.5

NC = 2
NS = 16
NW = NC * NS
EPT = E // NW
CHUNK = 80
NCHUNK = EPT // CHUNK
NG = 5
GC = NCHUNK // NG
NGD = 25
GCD = NCHUNK // NGD
FSL = D // 16
CP = N // NS - 1
TAIL = N - NS * CP
NBLK = N // CHUNK
ZPASS = -(-NBLK // NS)

_mesh = plsc.VectorSubcoreMesh(core_axis_name="c", subcore_axis_name="s")


def _degs_body(src_hbm, dst_hbm, w_hbm, degr_hbm, degc_hbm,
               idx, w_l, rows, degsh):
    c = lax.axis_index("c")
    s = lax.axis_index("s")
    wid = c * NS + s

    zero16 = jnp.zeros((16,), jnp.float32)

    def one_pass(edge_hbm, out_hbm):
        def _zrow(i, carry):
            for f in range(FSL):
                rows[i, pl.ds(f * 16, 16)] = zero16
            return carry
        lax.fori_loop(0, CHUNK, _zrow, 0)

        def _zacc(t, carry):
            b = t * NS + s

            @pl.when(b < NBLK)
            def _do():
                pltpu.sync_copy(rows, degsh.at[pl.ds(b * CHUNK, CHUNK)])
            return carry
        lax.fori_loop(0, ZPASS, _zacc, 0)

        plsc.subcore_barrier()

        def _gloop(g, carry0):
            pltpu.sync_copy(edge_hbm.at[wid, g], idx)
            pltpu.sync_copy(w_hbm.at[wid, g], w_l)

            def _chunk(j, carry):
                def _group(gg, carry2):
                    wvec = w_l[j, pl.ds(gg * 16, 16)]
                    for i in range(16):
                        k = gg * 16 + i
                        rows[k, pl.ds(0, 16)] = jnp.full(
                            (16,), wvec[i], dtype=jnp.float32)
                    return carry2
                lax.fori_loop(0, CHUNK // 16, _group, 0)

                pltpu.sync_copy(rows, degsh.at[idx.at[j]], add=True)
                return carry
            lax.fori_loop(0, GC, _chunk, 0)
            return carry0
        lax.fori_loop(0, NG, _gloop, 0)

        plsc.subcore_barrier()

        pltpu.sync_copy(degsh.at[pl.ds(s * CP, CP)],
                        out_hbm.at[c, pl.ds(s * CP, CP)])

        @pl.when(s == 0)
        def _out_tail():
            pltpu.sync_copy(degsh.at[pl.ds(NS * CP, TAIL)],
                            out_hbm.at[c, pl.ds(NS * CP, TAIL)])

        plsc.subcore_barrier()

    one_pass(src_hbm, degr_hbm)
    one_pass(dst_hbm, degc_hbm)


_degs = functools.partial(
    pl.kernel,
    out_type=(jax.ShapeDtypeStruct((NC, N, D), jnp.float32),
              jax.ShapeDtypeStruct((NC, N, D), jnp.float32)),
    mesh=_mesh,
    scratch_types=[
        pltpu.VMEM((GC, CHUNK), jnp.int32),
        pltpu.VMEM((GC, CHUNK), jnp.float32),
        pltpu.VMEM((CHUNK, D), jnp.float32),
        pltpu.VMEM_SHARED((N, D), jnp.float32),
    ],
)(_degs_body)


def _spmm_body(src_hbm, dst_hbm, w_hbm, table_hbm, out_hbm,
               idx_s, idx_d, w_l, rows, acc, sem):
    c = lax.axis_index("c")
    s = lax.axis_index("s")
    wid = c * NS + s

    zero16 = jnp.zeros((16,), jnp.float32)

    def _zrow(i, carry):
        for f in range(FSL):
            rows[0, i, pl.ds(f * 16, 16)] = zero16
        return carry
    lax.fori_loop(0, CHUNK, _zrow, 0)

    def _zacc(t, carry):
        b = t * NS + s

        @pl.when(b < NBLK)
        def _do():
            pltpu.sync_copy(rows.at[0], acc.at[pl.ds(b * CHUNK, CHUNK)])
        return carry
    lax.fori_loop(0, ZPASS, _zacc, 0)

    plsc.subcore_barrier()

    def _gloop(g, carry0):
        pltpu.sync_copy(src_hbm.at[wid, g], idx_s)
        pltpu.sync_copy(dst_hbm.at[wid, g], idx_d)
        pltpu.sync_copy(w_hbm.at[wid, g], w_l)

        def _scale(p, j):
            @plsc.parallel_loop(0, CHUNK // 16)
            def _group(gg):
                wvec = w_l[j, pl.ds(gg * 16, 16)]
                for i in range(16):
                    k = gg * 16 + i
                    wv = jnp.full((16,), wvec[i], dtype=jnp.float32)
                    for f in range(FSL):
                        sl = pl.ds(f * 16, 16)
                        rows[p, k, sl] = rows[p, k, sl] * wv

        def _gath(p, j):
            return pltpu.make_async_copy(table_hbm.at[idx_s.at[j]],
                                         rows.at[p], sem.at[p])

        _gath(0, 0).start()

        def _pair(kk, carry):
            j0 = 2 * kk
            j1 = j0 + 1
            _gath(1, j1).start()
            _gath(0, j0).wait()
            _scale(0, j0)
            pltpu.sync_copy(rows.at[0], acc.at[idx_d.at[j0]], add=True)
            _gath(0, j0 + 2).start()
            _gath(1, j1).wait()
            _scale(1, j1)
            pltpu.sync_copy(rows.at[1], acc.at[idx_d.at[j1]], add=True)
            return carry
        lax.fori_loop(0, (GC - 1) // 2, _pair, 0)

        _gath(0, GC - 1).wait()
        _scale(0, GC - 1)
        pltpu.sync_copy(rows.at[0], acc.at[idx_d.at[GC - 1]], add=True)
        return carry0
    lax.fori_loop(0, NG, _gloop, 0)

    plsc.subcore_barrier()

    pltpu.sync_copy(acc.at[pl.ds(s * CP, CP)], out_hbm.at[c, pl.ds(s * CP, CP)])

    @pl.when(s == 0)
    def _out_tail():
        pltpu.sync_copy(acc.at[pl.ds(NS * CP, TAIL)],
                        out_hbm.at[c, pl.ds(NS * CP, TAIL)])


_spmm = functools.partial(
    pl.kernel,
    out_type=jax.ShapeDtypeStruct((NC, N, D), jnp.float32),
    mesh=_mesh,
    scratch_types=[
        pltpu.VMEM((GC, CHUNK), jnp.int32),
        pltpu.VMEM((GC, CHUNK), jnp.int32),
        pltpu.VMEM((GC, CHUNK), jnp.float32),
        pltpu.VMEM((2, CHUNK, D), jnp.float32),
        pltpu.VMEM_SHARED((N, D), jnp.float32),
        pltpu.SemaphoreType.DMA((2,)),
    ],
)(_spmm_body)


BR = 1000
GRID = N // BR

_PREC = lax.Precision.HIGHEST


def _tc_front_kern(x_ref, ax0_ref, ax1_ref, dg_ref, dinv_ref,
                   W0_ref, b0_ref, Whp_ref, bhp_ref,
                   h_ref, lx_ref, hs_ref):
    x = x_ref[...]
    h = jnp.maximum(
        jnp.dot(x, W0_ref[...], precision=_PREC,
                preferred_element_type=jnp.float32) + b0_ref[...], 0.0)
    h_ref[...] = h
    hs_ref[...] = dinv_ref[...] * h
    t = dg_ref[...] * x - ax0_ref[...] - ax1_ref[...]
    lx_ref[...] = jnp.maximum(
        jnp.dot(t, Whp_ref[...], precision=_PREC,
                preferred_element_type=jnp.float32) + bhp_ref[...], 0.0)


def _tc_front(x, ax0, ax1, dg, dinv2d, W0, b0, Whp, bhp):
    row_spec = pl.BlockSpec((BR, D), lambda i: (i, 0))
    col_spec = pl.BlockSpec((BR, 1), lambda i: (i, 0))
    return pl.pallas_call(
        _tc_front_kern,
        grid=(GRID,),
        in_specs=[row_spec, row_spec, row_spec, col_spec, col_spec,
                  pl.BlockSpec((D, H), lambda i: (0, 0)),
                  pl.BlockSpec((1, H), lambda i: (0, 0)),
                  pl.BlockSpec((D, H), lambda i: (0, 0)),
                  pl.BlockSpec((1, H), lambda i: (0, 0))],
        out_specs=[pl.BlockSpec((BR, H), lambda i: (i, 0)),
                   pl.BlockSpec((BR, H), lambda i: (i, 0)),
                   pl.BlockSpec((BR, H), lambda i: (i, 0))],
        out_shape=[jax.ShapeDtypeStruct((N, H), jnp.float32),
                   jax.ShapeDtypeStruct((N, H), jnp.float32),
                   jax.ShapeDtypeStruct((N, H), jnp.float32)],
    )(x, ax0, ax1, dg, dinv2d, W0, b0, Whp, bhp)


def _tc_layer_kern(beta, agg0_ref, agg1_ref, h_ref, h0_ref, lx_ref,
                   dinv_ref, Wg_ref, bg_ref, out_ref, outs_ref):
    dv = dinv_ref[...]
    agg = dv * (agg0_ref[...] + agg1_ref[...]) + dv * dv * h_ref[...]
    support = (ALPHA * agg + (1.0 - ALPHA) * h0_ref[...]
               + GAMMA * lx_ref[...])
    hn = jnp.maximum(
        (1.0 - beta) * support
        + beta * (jnp.dot(support, Wg_ref[...], precision=_PREC,
                          preferred_element_type=jnp.float32) + bg_ref[...]),
        0.0)
    out_ref[...] = hn
    outs_ref[...] = dv * hn


def _tc_layer(beta, agg0, agg1, h, h0, lx, dinv2d, Wgi, bgi):
    row_spec = pl.BlockSpec((BR, H), lambda i: (i, 0))
    return pl.pallas_call(
        functools.partial(_tc_layer_kern, beta),
        grid=(GRID,),
        in_specs=[row_spec, row_spec, row_spec, row_spec, row_spec,
                  pl.BlockSpec((BR, 1), lambda i: (i, 0)),
                  pl.BlockSpec((H, H), lambda i: (0, 0)),
                  pl.BlockSpec((1, H), lambda i: (0, 0))],
        out_specs=[row_spec, row_spec],
        out_shape=[jax.ShapeDtypeStruct((N, H), jnp.float32),
                   jax.ShapeDtypeStruct((N, H), jnp.float32)],
    )(agg0, agg1, h, h0, lx, dinv2d, Wgi, bgi)


def _tc_out_kern(h_ref, Wout_ref, bout_ref, o_ref):
    z = jnp.dot(h_ref[...], Wout_ref[...], precision=_PREC,
                preferred_element_type=jnp.float32) + bout_ref[...]
    m = jnp.max(z, axis=1, keepdims=True)
    zs = z - m
    o_ref[...] = zs - jnp.log(jnp.sum(jnp.exp(zs), axis=1, keepdims=True))


def _tc_out(h, Wout, bout):
    return pl.pallas_call(
        _tc_out_kern,
        grid=(GRID,),
        in_specs=[pl.BlockSpec((BR, H), lambda i: (i, 0)),
                  pl.BlockSpec((H, C), lambda i: (0, 0)),
                  pl.BlockSpec((1, C), lambda i: (0, 0))],
        out_specs=pl.BlockSpec((BR, C), lambda i: (i, 0)),
        out_shape=jax.ShapeDtypeStruct((N, C), jnp.float32),
    )(h, Wout, bout)


def kernel(x, edge_index, edge_weight, W0, b0, Whp, bhp, Wg, bg, Wout, bout):
    row = edge_index[0]
    col = edge_index[1]
    src3 = row.reshape(NW, NG, GC, CHUNK)
    dst3 = col.reshape(NW, NG, GC, CHUNK)
    w3 = edge_weight.reshape(NW, NG, GC, CHUNK)
    axp = _spmm(src3, dst3, w3, x)
    degr, degc = _degs(src3, dst3, w3)
    deg_row = (degr[0, :, 0] + degr[1, :, 0]).reshape(N, 1)
    dinv = lax.rsqrt(degc[0, :, 0] + degc[1, :, 0] + 1.0)
    dinv2d = dinv.reshape(N, 1)
    h, lx, hs = _tc_front(x, axp[0], axp[1], deg_row, dinv2d,
                          W0, b0.reshape(1, H), Whp, bhp.reshape(1, H))
    h0 = h
    for i in range(NLAYER):
        beta = LAMDA / (i + 2)
        aggp = _spmm(src3, dst3, w3, hs)
        h, hs = _tc_layer(beta, aggp[0], aggp[1], h, h0, lx, dinv2d,
                          Wg[i], bg[i].reshape(1, H))
    return _tc_out(h, Wout, bout.reshape(1, C))
